# baseline (speedup 1.0000x reference)
import functools

import numpy as np

import jax
import jax.numpy as jnp
from jax import lax
from jax.experimental import pallas as pl
from jax.experimental.pallas import tpu as pltpu
from jax.experimental.pallas import tpu_sc as plsc


def _pack_perm(d):
    perm = np.empty(d, np.int32)
    for c0 in range(0, d, 32):
        for i in range(16):
            perm[c0 + 2 * i] = c0 + i
            perm[c0 + 2 * i + 1] = c0 + 16 + i
    return perm


def _sc_gather_bf16(table, idx, chunk=32):
    n_tokens = idx.shape[0]
    d = table.shape[1]
    groups = d // 16
    info = plsc.get_sparse_core_info()
    num_workers = info.num_cores * info.num_subcores
    per_worker = n_tokens // num_workers
    n_chunks = per_worker // chunk
    mesh = plsc.VectorSubcoreMesh(core_axis_name="c", subcore_axis_name="s")

    @functools.partial(
        pl.kernel,
        mesh=mesh,
        out_type=jax.ShapeDtypeStruct((n_tokens * d // 2,), jnp.int32),
        scratch_types=[
            pltpu.VMEM((per_worker,), jnp.int32),
            pltpu.VMEM((chunk, d), jnp.float32),
            pltpu.VMEM((chunk, d), jnp.float32),
            pltpu.VMEM((chunk * d // 2,), jnp.int32),
            pltpu.VMEM((chunk * d // 2,), jnp.int32),
            pltpu.SemaphoreType.DMA,
            pltpu.SemaphoreType.DMA,
            pltpu.SemaphoreType.DMA,
            pltpu.SemaphoreType.DMA,
        ],
    )
    def gather_kernel(table_hbm, idx_hbm, out_hbm, idx_v, ra, rb, ba, bb,
                      gsa, gsb, osa, osb):
        wid = lax.axis_index("s") * info.num_cores + lax.axis_index("c")
        base = wid * per_worker
        pltpu.sync_copy(idx_hbm.at[pl.ds(base, per_worker)], idx_v)
        rows = (ra, rb)
        bfs = (ba, bb)
        gsems = (gsa, gsb)
        osems = (osa, osb)

        unroll = 4
        rnd = jnp.int32(0x8000)
        himask = jnp.int32(-65536)

        def convert(rv, bv):
            def row_body(r, carry):
                rbase = r * (d // 2)

                def col_body(j, c2):
                    cbase = j * (32 * unroll)
                    for u in range(unroll):
                        off = cbase + u * 32
                        a = rv[r, pl.ds(off, 16)]
                        b = rv[r, pl.ds(off + 16, 16)]
                        ai = lax.bitcast_convert_type(a, jnp.int32) + rnd
                        bi = lax.bitcast_convert_type(b, jnp.int32) + rnd
                        bv[pl.ds(rbase + off // 2, 16)] = (
                            lax.shift_right_logical(ai, 16) | (bi & himask))
                    return c2
                return lax.fori_loop(0, d // (32 * unroll), col_body, carry)
            lax.fori_loop(0, chunk, row_body, 0)

        def start_gather(i):
            return pltpu.async_copy(
                table_hbm.at[idx_v.at[pl.ds(i * chunk, chunk)]],
                rows[i % 2], gsems[i % 2])

        out_handles = [None, None]
        h = start_gather(0)
        for i in range(n_chunks):
            h.wait()
            if i + 1 < n_chunks:
                h = start_gather(i + 1)
            if out_handles[i % 2] is not None:
                out_handles[i % 2].wait()
            convert(rows[i % 2], bfs[i % 2])
            out_handles[i % 2] = pltpu.async_copy(
                bfs[i % 2],
                out_hbm.at[pl.ds(
                    pl.multiple_of((base + i * chunk) * (d // 2), 8),
                    chunk * d // 2)],
                osems[i % 2])
        for oh in out_handles:
            if oh is not None:
                oh.wait()

    return gather_kernel(table, idx)


def _tc_project(g, sv, sw, pw, block_t=1024):
    n_tokens, d_embed = g.shape
    d_proj = pw.shape[0]
    vec_len = sv.shape[1]
    scale = float(d_proj) ** 0.5

    def body(g_ref, sv_ref, sw_ref, pw_ref, o_ref):
        e = g_ref[...].astype(jnp.float32) + lax.dot_general(
            sv_ref[...], sw_ref[...],
            (((1,), (0,)), ((), ())),
            preferred_element_type=jnp.float32,
        )
        o_ref[...] = lax.dot_general(
            e.astype(jnp.bfloat16), pw_ref[...].astype(jnp.bfloat16),
            (((1,), (1,)), ((), ())),
            preferred_element_type=jnp.float32,
        ) * scale

    return pl.pallas_call(
        body,
        grid=(n_tokens // block_t,),
        in_specs=[
            pl.BlockSpec((block_t, d_embed), lambda i: (i, 0)),
            pl.BlockSpec((block_t, vec_len), lambda i: (i, 0)),
            pl.BlockSpec((vec_len, d_embed), lambda i: (0, 0)),
            pl.BlockSpec((d_proj, d_embed), lambda i: (0, 0)),
        ],
        out_specs=pl.BlockSpec((block_t, d_proj), lambda i: (i, 0)),
        out_shape=jax.ShapeDtypeStruct((n_tokens, d_proj), jnp.float32),
    )(g, sv, sw, pw)


def kernel(inp, status_vec, emb_weight, status_weight, proj_W):
    b, l = inp.shape
    n_tokens = b * l
    d_embed = emb_weight.shape[1]
    d_proj = proj_W.shape[0]
    idx = inp.reshape(n_tokens).astype(jnp.int32)
    sv = status_vec.reshape(n_tokens, status_vec.shape[-1])

    perm = _pack_perm(d_embed)
    sw_p = status_weight[:, perm]
    pw_p = proj_W[:, perm]
    g_i32 = _sc_gather_bf16(emb_weight, idx)
    g = lax.bitcast_convert_type(g_i32, jnp.bfloat16).reshape(
        n_tokens, d_embed)
    out = _tc_project(g, sv, sw_p, pw_p)
    return out.reshape(b, l, d_proj)


# speedup vs baseline: 2.6196x; 2.6196x over previous
import functools

import numpy as np

import jax
import jax.numpy as jnp
from jax import lax
from jax.experimental import pallas as pl
from jax.experimental.pallas import tpu as pltpu
from jax.experimental.pallas import tpu_sc as plsc


def _pack_perm(d):
    perm = np.empty(d, np.int32)
    for c0 in range(0, d, 32):
        for i in range(16):
            perm[c0 + 2 * i] = c0 + i
            perm[c0 + 2 * i + 1] = c0 + 16 + i
    return perm


def _sc_gather_bf16(table, idx, chunk=32):
    n_tokens = idx.shape[0]
    d = table.shape[1]
    groups = d // 16
    info = plsc.get_sparse_core_info()
    num_workers = info.num_cores * info.num_subcores
    per_worker = n_tokens // num_workers
    n_chunks = per_worker // chunk
    mesh = plsc.VectorSubcoreMesh(core_axis_name="c", subcore_axis_name="s")

    @functools.partial(
        pl.kernel,
        mesh=mesh,
        out_type=jax.ShapeDtypeStruct((n_tokens * d // 2,), jnp.int32),
        scratch_types=[
            pltpu.VMEM((per_worker,), jnp.int32),
            pltpu.VMEM((chunk, d), jnp.float32),
            pltpu.VMEM((chunk, d), jnp.float32),
            pltpu.VMEM((chunk * d // 2,), jnp.int32),
            pltpu.VMEM((chunk * d // 2,), jnp.int32),
            pltpu.SemaphoreType.DMA,
            pltpu.SemaphoreType.DMA,
            pltpu.SemaphoreType.DMA,
            pltpu.SemaphoreType.DMA,
        ],
    )
    def gather_kernel(table_hbm, idx_hbm, out_hbm, idx_v, ra, rb, ba, bb,
                      gsa, gsb, osa, osb):
        wid = lax.axis_index("s") * info.num_cores + lax.axis_index("c")
        base = wid * per_worker
        pltpu.sync_copy(idx_hbm.at[pl.ds(base, per_worker)], idx_v)
        rows = (ra, rb)
        bfs = (ba, bb)
        gsems = (gsa, gsb)
        osems = (osa, osb)

        rnd = jnp.int32(0x8000)
        himask = jnp.int32(-65536)
        half = d // 2
        gpr = half // 16

        def convert(rv, bv):
            @plsc.parallel_loop(0, chunk * gpr, step=1, unroll=8)
            def _(g):
                r = g >> 5
                cb = (g & (gpr - 1)) * 16
                a = rv[r, pl.ds(cb, 16)]
                b = rv[r, pl.ds(cb + half, 16)]
                ai = lax.bitcast_convert_type(a, jnp.int32) + rnd
                bi = lax.bitcast_convert_type(b, jnp.int32) + rnd
                bv[pl.ds(g * 16, 16)] = (
                    lax.shift_right_logical(ai, 16) | (bi & himask))

        def start_gather(i):
            return pltpu.async_copy(
                table_hbm.at[idx_v.at[pl.ds(i * chunk, chunk)]],
                rows[i % 2], gsems[i % 2])

        out_handles = [None, None]
        h = start_gather(0)
        for i in range(n_chunks):
            h.wait()
            if i + 1 < n_chunks:
                h = start_gather(i + 1)
            if out_handles[i % 2] is not None:
                out_handles[i % 2].wait()
            convert(rows[i % 2], bfs[i % 2])
            out_handles[i % 2] = pltpu.async_copy(
                bfs[i % 2],
                out_hbm.at[pl.ds(
                    pl.multiple_of((base + i * chunk) * (d // 2), 8),
                    chunk * d // 2)],
                osems[i % 2])
        for oh in out_handles:
            if oh is not None:
                oh.wait()

    return gather_kernel(table, idx)


def _tc_project(g, sv, sw, pw, block_t=1024):
    n_tokens = g.shape[0]
    d_embed = pw.shape[1]
    d_proj = pw.shape[0]
    vec_len = sv.shape[1]
    scale = float(d_proj) ** 0.5

    half = d_embed // 2

    def body(g_ref, sv_ref, sw_ref, pw_ref, o_ref):
        h = lax.dot_general(
            sv_ref[...], sw_ref[...],
            (((1,), (0,)), ((), ())),
            preferred_element_type=jnp.float32,
        )
        gi = g_ref[...]
        ga = lax.bitcast_convert_type(gi << 16, jnp.float32)
        gb = lax.bitcast_convert_type(gi & jnp.int32(-65536), jnp.float32)
        e1 = (ga + h[:, :half]).astype(jnp.bfloat16)
        e2 = (gb + h[:, half:]).astype(jnp.bfloat16)
        pw = pw_ref[...].astype(jnp.bfloat16)
        acc = lax.dot_general(
            e1, pw[:, :half],
            (((1,), (1,)), ((), ())),
            preferred_element_type=jnp.float32,
        ) + lax.dot_general(
            e2, pw[:, half:],
            (((1,), (1,)), ((), ())),
            preferred_element_type=jnp.float32,
        )
        o_ref[...] = acc * scale

    return pl.pallas_call(
        body,
        grid=(n_tokens // block_t,),
        in_specs=[
            pl.BlockSpec((block_t, d_embed // 2), lambda i: (i, 0)),
            pl.BlockSpec((block_t, vec_len), lambda i: (i, 0)),
            pl.BlockSpec((vec_len, d_embed), lambda i: (0, 0)),
            pl.BlockSpec((d_proj, d_embed), lambda i: (0, 0)),
        ],
        out_specs=pl.BlockSpec((block_t, d_proj), lambda i: (i, 0)),
        out_shape=jax.ShapeDtypeStruct((n_tokens, d_proj), jnp.float32),
    )(g, sv, sw, pw)


def kernel(inp, status_vec, emb_weight, status_weight, proj_W):
    b, l = inp.shape
    n_tokens = b * l
    d_embed = emb_weight.shape[1]
    d_proj = proj_W.shape[0]
    idx = inp.reshape(n_tokens).astype(jnp.int32)
    sv = status_vec.reshape(n_tokens, status_vec.shape[-1])

    g_i32 = _sc_gather_bf16(emb_weight, idx).reshape(
        n_tokens, d_embed // 2)
    out = _tc_project(g_i32, sv, status_weight, proj_W)
    return out.reshape(b, l, d_proj)
